# baseline (speedup 1.0000x reference)
import functools

import jax
import jax.numpy as jnp
from jax import lax
from jax.experimental import pallas as pl
from jax.experimental.pallas import tpu as pltpu
from jax.experimental.pallas import tpu_sc as plsc

_N = 256
_NC = 2
_NS = 16
_NW = _NC * _NS
_ROWS = 4096
_ROW_W = 4096
_RPW = _ROWS // _NW
_CHUNK = 8
_NCHUNK = _RPW // _CHUNK


_NBUF = 3


def _sc_gather_body(x_hbm, idx_hbm, out_hbm, idx_v, buf_a, buf_b, buf_c,
                    gsem_a, gsem_b, gsem_c, ssem_a, ssem_b, ssem_c):
    wid = lax.axis_index("s") * _NC + lax.axis_index("c")
    base = wid * _RPW
    pltpu.sync_copy(idx_hbm.at[pl.ds(base, _RPW)], idx_v)

    bufs = (buf_a, buf_b, buf_c)
    gsems = (gsem_a, gsem_b, gsem_c)
    ssems = (ssem_a, ssem_b, ssem_c)

    def gather(c):
        return pltpu.async_copy(
            x_hbm.at[idx_v.at[pl.ds(c * _CHUNK, _CHUNK)]],
            bufs[c % _NBUF], gsems[c % _NBUF])

    def scatter(c):
        return pltpu.async_copy(
            bufs[c % _NBUF], out_hbm.at[pl.ds(base + c * _CHUNK, _CHUNK)],
            ssems[c % _NBUF])

    gs = [None] * _NCHUNK
    ss = [None] * _NCHUNK
    for c in range(min(_NBUF, _NCHUNK)):
        gs[c] = gather(c)
    for c in range(_NCHUNK):
        gs[c].wait()
        ss[c] = scatter(c)
        n = c + _NBUF
        if n < _NCHUNK:
            ss[n - _NBUF].wait()
            gs[n] = gather(n)
    for c in range(max(0, _NCHUNK - _NBUF), _NCHUNK):
        ss[c].wait()


def _sc_gather(x16, idx16):
    mesh = plsc.VectorSubcoreMesh(core_axis_name="c", subcore_axis_name="s")
    run = functools.partial(
        pl.kernel, mesh=mesh,
        out_type=jax.ShapeDtypeStruct((_ROWS, _ROW_W), jnp.float32),
        scratch_types=[
            pltpu.VMEM((_RPW,), jnp.int32),
            pltpu.VMEM((_CHUNK, _ROW_W), jnp.float32),
            pltpu.VMEM((_CHUNK, _ROW_W), jnp.float32),
            pltpu.VMEM((_CHUNK, _ROW_W), jnp.float32),
            pltpu.SemaphoreType.DMA,
            pltpu.SemaphoreType.DMA,
            pltpu.SemaphoreType.DMA,
            pltpu.SemaphoreType.DMA,
            pltpu.SemaphoreType.DMA,
            pltpu.SemaphoreType.DMA,
        ],
    )(_sc_gather_body)
    return run(x16, idx16)


def _tc_body(y_smem, y_col, x_seq, out1, out2, p_ref):
    j = pl.program_id(0)

    @pl.when(j == 0)
    def _():
        iota_k = jax.lax.broadcasted_iota(jnp.int32, (_N, _N), 1)
        p_ref[...] = (y_col[...] == iota_k).astype(jnp.float32)

    xs = x_seq[0]
    p = p_ref[...]
    out1[0] = jax.lax.dot_general(
        p, xs, (((1,), (0,)), ((), ())),
        preferred_element_type=jnp.float32,
        precision=jax.lax.Precision.DEFAULT)
    out2[0] = jax.lax.dot_general(
        xs, p, (((1,), (1,)), ((), ())),
        preferred_element_type=jnp.float32,
        precision=jax.lax.Precision.DEFAULT)


def _tc_matmuls(x, y32):
    y_col = y32.reshape(_N, 1)
    grid_spec = pltpu.PrefetchScalarGridSpec(
        num_scalar_prefetch=1,
        grid=(_N,),
        in_specs=[
            pl.BlockSpec((_N, 1), lambda j, y_ref: (0, 0)),
            pl.BlockSpec((1, _N, _N), lambda j, y_ref: (j, 0, 0)),
        ],
        out_specs=[
            pl.BlockSpec((1, _N, _N), lambda j, y_ref: (j, 0, 0)),
            pl.BlockSpec((1, _N, _N), lambda j, y_ref: (j, 0, 0)),
        ],
        scratch_shapes=[pltpu.VMEM((_N, _N), jnp.float32)],
    )
    out_shape = [jax.ShapeDtypeStruct((_N, _N, _N), jnp.float32)] * 2
    return pl.pallas_call(
        _tc_body, grid_spec=grid_spec, out_shape=out_shape,
    )(y32, y_col, x)


def kernel(x, y):
    y32 = y.astype(jnp.int32)
    rpp = _ROWS // _N
    idx16 = (y32[:, None] * rpp + jnp.arange(rpp, dtype=jnp.int32)).reshape(-1)
    x16 = x.reshape(_ROWS, _ROW_W)
    out0 = _sc_gather(x16, idx16).reshape(_N, _N, _N)
    out1, out2 = _tc_matmuls(x, y32)
    return (out0, out1, out2)


# speedup vs baseline: 1.6411x; 1.6411x over previous
import functools

import jax
import jax.numpy as jnp
from jax import lax
from jax.experimental import pallas as pl
from jax.experimental.pallas import tpu as pltpu
from jax.experimental.pallas import tpu_sc as plsc

_N = 256
_NC = 2
_NS = 16
_NW = _NC * _NS
_ROWS = _N * _N
_RPW = _ROWS // _NW
_CHUNK = 128
_NCHUNK = _RPW // _CHUNK
_NBUF = 3


def _sc_gather_body(x_hbm, idx_hbm, out_hbm, idx_v, buf_a, buf_b, buf_c,
                    gsem_a, gsem_b, gsem_c, ssem_a, ssem_b, ssem_c):
    wid = lax.axis_index("s") * _NC + lax.axis_index("c")
    base = wid * _RPW
    pltpu.sync_copy(idx_hbm.at[pl.ds(base, _RPW)], idx_v)

    bufs = (buf_a, buf_b, buf_c)
    gsems = (gsem_a, gsem_b, gsem_c)
    ssems = (ssem_a, ssem_b, ssem_c)

    def gather(c):
        return pltpu.async_copy(
            x_hbm.at[idx_v.at[pl.ds(c * _CHUNK, _CHUNK)]],
            bufs[c % _NBUF], gsems[c % _NBUF])

    def scatter(c):
        return pltpu.async_copy(
            bufs[c % _NBUF], out_hbm.at[pl.ds(base + c * _CHUNK, _CHUNK)],
            ssems[c % _NBUF])

    gs = [None] * _NCHUNK
    ss = [None] * _NCHUNK
    for c in range(min(_NBUF, _NCHUNK)):
        gs[c] = gather(c)
    for c in range(_NCHUNK):
        gs[c].wait()
        ss[c] = scatter(c)
        n = c + _NBUF
        if n < _NCHUNK:
            ss[n - _NBUF].wait()
            gs[n] = gather(n)
    for c in range(max(0, _NCHUNK - _NBUF), _NCHUNK):
        ss[c].wait()


def _sc_gather(x2, idx):
    mesh = plsc.VectorSubcoreMesh(core_axis_name="c", subcore_axis_name="s")
    run = functools.partial(
        pl.kernel, mesh=mesh,
        out_type=jax.ShapeDtypeStruct((_ROWS, _N), jnp.float32),
        scratch_types=[
            pltpu.VMEM((_RPW,), jnp.int32),
            pltpu.VMEM((_CHUNK, _N), jnp.float32),
            pltpu.VMEM((_CHUNK, _N), jnp.float32),
            pltpu.VMEM((_CHUNK, _N), jnp.float32),
            pltpu.SemaphoreType.DMA,
            pltpu.SemaphoreType.DMA,
            pltpu.SemaphoreType.DMA,
            pltpu.SemaphoreType.DMA,
            pltpu.SemaphoreType.DMA,
            pltpu.SemaphoreType.DMA,
        ],
    )(_sc_gather_body)
    return run(x2, idx)


def _tc_body(y_smem, y_col, x_seq, out1, out2, p_ref):
    j = pl.program_id(0)

    @pl.when(j == 0)
    def _():
        iota_k = jax.lax.broadcasted_iota(jnp.int32, (_N, _N), 1)
        p_ref[...] = (y_col[...] == iota_k).astype(jnp.float32)

    xs = x_seq[0]
    p = p_ref[...]
    out1[0] = jax.lax.dot_general(
        p, xs, (((1,), (0,)), ((), ())),
        preferred_element_type=jnp.float32,
        precision=jax.lax.Precision.DEFAULT)
    out2[0] = jax.lax.dot_general(
        xs, p, (((1,), (1,)), ((), ())),
        preferred_element_type=jnp.float32,
        precision=jax.lax.Precision.DEFAULT)


def _tc_matmuls(x, y32):
    y_col = y32.reshape(_N, 1)
    grid_spec = pltpu.PrefetchScalarGridSpec(
        num_scalar_prefetch=1,
        grid=(_N,),
        in_specs=[
            pl.BlockSpec((_N, 1), lambda j, y_ref: (0, 0)),
            pl.BlockSpec((1, _N, _N), lambda j, y_ref: (j, 0, 0)),
        ],
        out_specs=[
            pl.BlockSpec((1, _N, _N), lambda j, y_ref: (j, 0, 0)),
            pl.BlockSpec((1, _N, _N), lambda j, y_ref: (j, 0, 0)),
        ],
        scratch_shapes=[pltpu.VMEM((_N, _N), jnp.float32)],
    )
    out_shape = [jax.ShapeDtypeStruct((_N, _N, _N), jnp.float32)] * 2
    return pl.pallas_call(
        _tc_body, grid_spec=grid_spec, out_shape=out_shape,
    )(y32, y_col, x)


def kernel(x, y):
    y32 = y.astype(jnp.int32)
    idx = (y32[:, None] * _N + jnp.arange(_N, dtype=jnp.int32)).reshape(-1)
    x2 = x.reshape(_ROWS, _N)
    out0 = _sc_gather(x2, idx).reshape(_N, _N, _N)
    out1, out2 = _tc_matmuls(x, y32)
    return (out0, out1, out2)
